# Initial kernel scaffold; baseline (speedup 1.0000x reference)
import jax
import jax.numpy as jnp
from jax.experimental import pallas as pl


def kernel(entity_ids, type_memberships):
    raise NotImplementedError("write your pallas kernel here")


# speedup vs baseline: 2.9849x; 2.9849x over previous
import functools

import jax
import jax.numpy as jnp
from jax import lax
from jax.experimental import pallas as pl
from jax.experimental.pallas import tpu as pltpu
from jax.experimental.pallas import tpu_sc as plsc


@functools.lru_cache(maxsize=None)
def _make_gather(V, D, B):
    info = plsc.get_sparse_core_info()
    NC, NS = info.num_cores, info.num_subcores
    NW = NC * NS
    assert B % NW == 0
    b_per_w = B // NW
    C = 128
    assert b_per_w % C == 0
    n_chunks = b_per_w // C

    mesh = plsc.VectorSubcoreMesh(core_axis_name="c", subcore_axis_name="s")

    @functools.partial(
        pl.kernel,
        mesh=mesh,
        out_type=jax.ShapeDtypeStruct((B, D), jnp.float32),
        scratch_types=[
            pltpu.VMEM((n_chunks, C), jnp.int32),
            pltpu.VMEM((C, D), jnp.float32),
            pltpu.SemaphoreType.DMA,
        ],
    )
    def k(idx_hbm, table_hbm, out_hbm, idx_v, rows_v, sem):
        wid = lax.axis_index("s") * NC + lax.axis_index("c")
        base = wid * b_per_w
        pltpu.sync_copy(idx_hbm.at[wid], idx_v)

        def body(i, carry):
            pltpu.async_copy(table_hbm.at[idx_v.at[i]], rows_v, sem).wait()
            pltpu.sync_copy(rows_v, out_hbm.at[pl.ds(base + i * C, C)])
            return carry

        lax.fori_loop(0, n_chunks, body, 0)

    def run(idx3, table):
        return k(idx3, table)

    return run


def kernel(entity_ids, type_memberships):
    batch, ncand = entity_ids.shape
    V, D = type_memberships.shape
    B = batch * ncand
    info = plsc.get_sparse_core_info()
    NW = info.num_cores * info.num_subcores
    C = 128
    idx3 = entity_ids.reshape(NW, (B // NW) // C, C).astype(jnp.int32)
    out = _make_gather(V, D, B)(idx3, type_memberships)
    return out.reshape(batch, ncand, D)


# speedup vs baseline: 3.3579x; 1.1250x over previous
import functools

import jax
import jax.numpy as jnp
from jax import lax
from jax.experimental import pallas as pl
from jax.experimental.pallas import tpu as pltpu
from jax.experimental.pallas import tpu_sc as plsc

_C = 128
_NBUF = 5
_K = 3


@functools.lru_cache(maxsize=None)
def _make_gather(V, D, B):
    info = plsc.get_sparse_core_info()
    NC, NS = info.num_cores, info.num_subcores
    NW = NC * NS
    assert B % (NW * _C) == 0
    b_per_w = B // NW
    n = b_per_w // _C
    assert n % _NBUF == 0

    mesh = plsc.VectorSubcoreMesh(core_axis_name="c", subcore_axis_name="s")

    @functools.partial(
        pl.kernel,
        mesh=mesh,
        out_type=jax.ShapeDtypeStruct((B, D), jnp.float32),
        scratch_types=(
            [pltpu.VMEM((n, _C), jnp.int32)]
            + [pltpu.VMEM((_C, D), jnp.float32) for _ in range(_NBUF)]
            + [pltpu.SemaphoreType.DMA for _ in range(2 * _NBUF)]
        ),
    )
    def k(idx_hbm, table_hbm, out_hbm, idx_v, *bufs_and_sems):
        bufs = bufs_and_sems[:_NBUF]
        gsem = bufs_and_sems[_NBUF:2 * _NBUF]
        ssem = bufs_and_sems[2 * _NBUF:]
        wid = lax.axis_index("s") * NC + lax.axis_index("c")
        base = wid * b_per_w
        pltpu.sync_copy(idx_hbm.at[wid], idx_v)

        def start_gather(i, b):
            pltpu.async_copy(table_hbm.at[idx_v.at[i]], bufs[b], gsem[b])

        def wait_gather(b):
            pltpu.make_async_copy(
                table_hbm.at[idx_v.at[0]], bufs[b], gsem[b]).wait()

        def start_store(i, b):
            pltpu.async_copy(bufs[b], out_hbm.at[pl.ds(base + i * _C, _C)],
                             ssem[b])

        def wait_store(b):
            pltpu.make_async_copy(
                bufs[b], out_hbm.at[pl.ds(base, _C)], ssem[b]).wait()

        for i in range(_K):
            start_gather(i, i)

        def body(g, carry):
            i0 = g * _NBUF
            for b in range(_NBUF):
                i = i0 + b
                bg = (b + _K) % _NBUF

                @pl.when(i + _K < n)
                def _():
                    @pl.when(i >= _NBUF - _K)
                    def _():
                        wait_store(bg)
                    start_gather(i + _K, bg)

                wait_gather(b)
                start_store(i, b)
            return carry

        lax.fori_loop(0, n // _NBUF, body, 0)

        wait_store((n - 2) % _NBUF)
        wait_store((n - 1) % _NBUF)

    def run(idx3, table):
        return k(idx3, table)

    return run


def kernel(entity_ids, type_memberships):
    batch, ncand = entity_ids.shape
    V, D = type_memberships.shape
    B = batch * ncand
    info = plsc.get_sparse_core_info()
    NW = info.num_cores * info.num_subcores
    idx3 = entity_ids.reshape(NW, (B // NW) // _C, _C).astype(jnp.int32)
    out = _make_gather(V, D, B)(idx3, type_memberships)
    return out.reshape(batch, ncand, D)


# speedup vs baseline: 6.0057x; 1.7885x over previous
import functools

import jax
import jax.numpy as jnp
from jax import lax
from jax.experimental import pallas as pl
from jax.experimental.pallas import tpu as pltpu
from jax.experimental.pallas import tpu_sc as plsc

_NBUF = 8
_K = 5


@functools.lru_cache(maxsize=None)
def _make_gather(V, D, batch, ncand):
    info = plsc.get_sparse_core_info()
    NC, NS = info.num_cores, info.num_subcores
    NW = NC * NS
    assert batch % NW == 0
    n = batch // NW
    assert n % _NBUF == 0

    mesh = plsc.VectorSubcoreMesh(core_axis_name="c", subcore_axis_name="s")

    @functools.partial(
        pl.kernel,
        mesh=mesh,
        out_type=jax.ShapeDtypeStruct((batch, ncand, D), jnp.float32),
        scratch_types=(
            [pltpu.VMEM((n, ncand), jnp.int32)]
            + [pltpu.VMEM((ncand, D), jnp.float32) for _ in range(_NBUF)]
            + [pltpu.SemaphoreType.DMA for _ in range(2 * _NBUF)]
        ),
    )
    def k(idx_hbm, table_hbm, out_hbm, idx_v, *bufs_and_sems):
        bufs = bufs_and_sems[:_NBUF]
        gsem = bufs_and_sems[_NBUF:2 * _NBUF]
        ssem = bufs_and_sems[2 * _NBUF:]
        wid = lax.axis_index("s") * NC + lax.axis_index("c")
        base = wid * n
        pltpu.sync_copy(idx_hbm.at[pl.ds(base, n)], idx_v)

        def start_gather(i, b):
            pltpu.async_copy(table_hbm.at[idx_v.at[i]], bufs[b], gsem[b])

        def wait_gather(b):
            pltpu.make_async_copy(
                table_hbm.at[idx_v.at[0]], bufs[b], gsem[b]).wait()

        def start_store(i, b):
            pltpu.async_copy(bufs[b], out_hbm.at[base + i], ssem[b])

        def wait_store(b):
            pltpu.make_async_copy(bufs[b], out_hbm.at[base], ssem[b]).wait()

        for i in range(_K):
            start_gather(i, i)

        def body(g, carry):
            i0 = g * _NBUF
            for b in range(_NBUF):
                i = i0 + b
                bg = (b + _K) % _NBUF

                @pl.when(i + _K < n)
                def _():
                    @pl.when(i >= _NBUF - _K)
                    def _():
                        wait_store(bg)
                    start_gather(i + _K, bg)

                wait_gather(b)
                start_store(i, b)
            return carry

        lax.fori_loop(0, n // _NBUF, body, 0)

        for j in range(_NBUF - _K):
            wait_store((n - 1 - j) % _NBUF)

    def run(idx, table):
        return k(idx, table)

    return run


def kernel(entity_ids, type_memberships):
    batch, ncand = entity_ids.shape
    V, D = type_memberships.shape
    return _make_gather(V, D, batch, ncand)(
        entity_ids.astype(jnp.int32), type_memberships)
